# baseline (speedup 1.0000x reference)
import functools
import jax
import jax.numpy as jnp
from jax import lax
from jax.experimental import pallas as pl
from jax.experimental.pallas import tpu as pltpu
from jax.experimental.pallas import tpu_sc as plsc

H = 8
D = 16
HD = H * D
ROW = HD + H
INV_SQRT_D = 0.25

NC = 2
NS = 16
NW = NC * NS
C = 32
NBUF = 2


def _cdiv(a, b):
    return (a + b - 1) // b


def _qkv_body(h_ref, wt_ref, q_ref, kv_ref):
    y = jnp.dot(h_ref[...], wt_ref[...], preferred_element_type=jnp.float32)
    q_ref[...] = y[:, :HD]
    kv_ref[...] = y[:, HD:]


def _qkv(h, wt, blk):
    n = h.shape[0]
    in_dim = h.shape[1]
    grid = n // blk
    return pl.pallas_call(
        _qkv_body,
        grid=(grid,),
        in_specs=[
            pl.BlockSpec((blk, in_dim), lambda i: (i, 0)),
            pl.BlockSpec((in_dim, 3 * HD), lambda i: (0, 0)),
        ],
        out_specs=[
            pl.BlockSpec((blk, HD), lambda i: (i, 0)),
            pl.BlockSpec((blk, 2 * HD), lambda i: (i, 0)),
        ],
        out_shape=[
            jax.ShapeDtypeStruct((n, HD), jnp.float32),
            jax.ShapeDtypeStruct((n, 2 * HD), jnp.float32),
        ],
    )(h, wt)


def _compute_chunk(kv_b, q_b, w_b):
    def _group(g, carry):
        lanes = lax.iota(jnp.int32, 16)
        rows = lanes + (g * 16)

        def _head(hh, carry2):
            col0 = hh * D
            acc = None
            for r in range(D):
                dcol = col0 + jnp.bitwise_and(lanes + r, 15)
                kvv = plsc.load_gather(kv_b, [rows, dcol])
                qv = plsc.load_gather(q_b, [rows, dcol])
                prod = kvv * qv
                acc = prod if acc is None else acc + prod
            sc = acc * INV_SQRT_D
            sc = jnp.minimum(jnp.maximum(sc, -5.0), 5.0)
            p = jnp.exp(sc)
            plsc.store_scatter(
                w_b, [rows, jnp.full((16,), HD + hh, jnp.int32)], p)
            for r in range(D):
                dcol = col0 + jnp.bitwise_and(lanes + r, 15)
                vv = plsc.load_gather(kv_b, [rows, HD + dcol])
                plsc.store_scatter(w_b, [rows, dcol], vv * p)
            return carry2

        lax.fori_loop(0, H, _head, 0, unroll=4)
        return carry

    lax.fori_loop(0, C // 16, _group, 0, unroll=2)


def _edge_body(acc_rows, ch,
               q_hbm, kv_hbm, pk_hbm, zeros_hbm, out_hbm,
               acc_s, pkblk, kvbuf, qbuf, wbuf, gidx, sidx, sems):
    c = lax.axis_index("c")
    s = lax.axis_index("s")
    w = s * NC + c

    sem_kv = [sems.at[0, b] for b in range(NBUF)]
    sem_q = [sems.at[1, b] for b in range(NBUF)]
    sem_w = [sems.at[2, b] for b in range(NBUF)]

    rpt = acc_rows // NS
    r0 = s * rpt
    pltpu.sync_copy(zeros_hbm.at[pl.ds(r0, rpt)], acc_s.at[pl.ds(r0, rpt)])

    row0 = w * ch
    pltpu.sync_copy(pk_hbm.at[pl.ds(row0, ch + NBUF)], pkblk)

    plsc.subcore_barrier()

    def _unpack_gather_idx(slot, local_row):
        for g in range(C // 16):
            pkv = pkblk[local_row, pl.ds(g * 16, 16)]
            gidx[0, slot, pl.ds(g * 16, 16)] = jnp.bitwise_and(
                pkv, jnp.int32(0xFFFF))
            gidx[1, slot, pl.ds(g * 16, 16)] = lax.shift_right_logical(
                pkv, jnp.int32(16))

    def _issue(slot, local_row):
        _unpack_gather_idx(slot, local_row)
        pltpu.async_copy(
            kv_hbm.at[gidx.at[0, slot]], kvbuf.at[slot], sem_kv[slot])
        pltpu.async_copy(
            q_hbm.at[gidx.at[1, slot]], qbuf.at[slot], sem_q[slot])

    def _drain(slot):
        pltpu.make_async_copy(
            kv_hbm.at[gidx.at[0, slot]], kvbuf.at[slot],
            sem_kv[slot]).wait()
        pltpu.make_async_copy(
            q_hbm.at[gidx.at[1, slot]], qbuf.at[slot], sem_q[slot]).wait()

    def _wait_scatter(slot):
        pltpu.make_async_copy(
            wbuf.at[slot], acc_s.at[sidx.at[slot]], sem_w[slot]).wait()

    for b in range(NBUF):
        _issue(b, b)

    def _round(t, carry):
        jj = NBUF * t
        for b in range(NBUF):
            _drain(b)

            @pl.when(t > 0)
            def _():
                _wait_scatter(b)

            _compute_chunk(kvbuf.at[b], qbuf.at[b], wbuf.at[b])
            for g in range(C // 16):
                pkv = pkblk[jj + b, pl.ds(g * 16, 16)]
                sidx[b, pl.ds(g * 16, 16)] = lax.shift_right_logical(
                    pkv, jnp.int32(16))
            pltpu.async_copy(
                wbuf.at[b], acc_s.at[sidx.at[b]], sem_w[b], add=True)
            _issue(b, jj + b + NBUF)
        return carry

    lax.fori_loop(0, ch // NBUF, _round, 0)

    for b in range(NBUF):
        _drain(b)
        _wait_scatter(b)

    plsc.subcore_barrier()
    pltpu.sync_copy(acc_s.at[pl.ds(r0, rpt)], out_hbm.at[c, pl.ds(r0, rpt)])


def _edge_phase(q, kv, pk2, zeros, acc_rows, ch):
    mesh = plsc.VectorSubcoreMesh(core_axis_name="c", subcore_axis_name="s")
    body = functools.partial(_edge_body, acc_rows, ch)
    return pl.kernel(
        body,
        out_type=jax.ShapeDtypeStruct((NC, acc_rows, ROW), jnp.float32),
        mesh=mesh,
        scratch_types=[
            pltpu.VMEM_SHARED((acc_rows, ROW), jnp.float32),
            pltpu.VMEM((ch + NBUF, C), jnp.int32),
            pltpu.VMEM((NBUF, C, 2 * HD), jnp.float32),
            pltpu.VMEM((NBUF, C, HD), jnp.float32),
            pltpu.VMEM((NBUF, C, ROW), jnp.float32),
            pltpu.VMEM((2, NBUF, C), jnp.int32),
            pltpu.VMEM((NBUF, C), jnp.int32),
            pltpu.SemaphoreType.DMA((3, NBUF)),
        ],
        compiler_params=pltpu.CompilerParams(
            use_tc_tiling_on_sc=False, needs_layout_passes=False),
    )(q, kv, pk2, zeros)


def _combine_body(acc_ref, out_ref):
    ab = acc_ref[...]
    a = ab[0] + ab[1]
    wv = a[:, :HD]
    z = a[:, HD:HD + H]
    rowi = lax.broadcasted_iota(jnp.int32, (H, HD), 0)
    coli = lax.broadcasted_iota(jnp.int32, (H, HD), 1)
    bmat = (coli // D == rowi).astype(jnp.float32)
    zrep = jnp.dot(z, bmat, preferred_element_type=jnp.float32)
    out_ref[...] = wv / zrep


def _combine(acc, n_out, blk):
    grid = n_out // blk
    return pl.pallas_call(
        _combine_body,
        grid=(grid,),
        in_specs=[pl.BlockSpec((NC, blk, ROW), lambda i: (0, i, 0))],
        out_specs=pl.BlockSpec((blk, HD), lambda i: (i, 0)),
        out_shape=jax.ShapeDtypeStruct((n_out, HD), jnp.float32),
    )(acc)


def kernel(h, edge_index, WQ, WK, WV):
    n, in_dim = h.shape
    e = edge_index.shape[1]

    wt = jnp.concatenate([WQ.T, WK.T, WV.T], axis=1)
    blk_n = 1000 if n % 1000 == 0 else 8
    n_pad = _cdiv(n, blk_n) * blk_n
    h_p = h if n_pad == n else jnp.pad(h, ((0, n_pad - n), (0, 0)))

    src = edge_index[0]
    dst = edge_index[1]
    ch = NBUF * _cdiv(e, NW * C * NBUF)
    rows_total = NW * ch + NBUF
    e_pad = rows_total * C
    src = jnp.concatenate([src, jnp.zeros((e_pad - e,), jnp.int32)])
    dst = jnp.concatenate([dst, jnp.full((e_pad - e,), n, jnp.int32)])
    pk = jnp.bitwise_or(src, jnp.left_shift(dst, 16))
    pk2 = pk.reshape(rows_total, C)

    acc_rows = max(n + 1, _cdiv(n, 1000) * 1000)
    acc_rows = NS * _cdiv(acc_rows, NS)
    zeros = jnp.zeros((acc_rows, ROW), jnp.float32)

    q, kv = _qkv(h_p, wt, blk_n)
    q = q[:n] if n_pad != n else q
    kv = kv[:n] if n_pad != n else kv
    acc = _edge_phase(q, kv, pk2, zeros, acc_rows, ch)
    blk_o = 1000 if n % 1000 == 0 else 8
    n_out = _cdiv(n, blk_o) * blk_o
    out = _combine(acc, n_out, blk_o)
    return out[:n].reshape(n, H, D)


# speedup vs baseline: 1.5110x; 1.5110x over previous
import functools
import jax
import jax.numpy as jnp
from jax import lax
from jax.experimental import pallas as pl
from jax.experimental.pallas import tpu as pltpu
from jax.experimental.pallas import tpu_sc as plsc

H = 8
D = 16
HD = H * D
ROW = HD + H
INV_SQRT_D = 0.25

NC = 2
NS = 16
NW = NC * NS
C = 32
NBUF = 2


def _cdiv(a, b):
    return (a + b - 1) // b


def _qkv_body(h_ref, wt_ref, q_ref, kv_ref):
    y = jnp.dot(h_ref[...], wt_ref[...], preferred_element_type=jnp.float32)
    q_ref[...] = y[:, :HD]
    kv_ref[...] = y[:, HD:]


def _qkv(h, wt, blk):
    n = h.shape[0]
    in_dim = h.shape[1]
    grid = n // blk
    return pl.pallas_call(
        _qkv_body,
        grid=(grid,),
        in_specs=[
            pl.BlockSpec((blk, in_dim), lambda i: (i, 0)),
            pl.BlockSpec((in_dim, 3 * HD), lambda i: (0, 0)),
        ],
        out_specs=[
            pl.BlockSpec((blk, HD), lambda i: (i, 0)),
            pl.BlockSpec((blk, 2 * HD), lambda i: (i, 0)),
        ],
        out_shape=[
            jax.ShapeDtypeStruct((n, HD), jnp.float32),
            jax.ShapeDtypeStruct((n, 2 * HD), jnp.float32),
        ],
    )(h, wt)


def _compute_chunk(kv_b, q_b, w_b):
    def _group(g, carry):
        lanes = lax.iota(jnp.int32, 16)
        rows = lanes + (g * 16)

        def _head(hh, carry2):
            col0 = hh * D
            acc = None
            for r in range(D):
                dcol = col0 + jnp.bitwise_xor(lanes, r)
                kvv = plsc.load_gather(kv_b, [rows, dcol])
                qv = plsc.load_gather(q_b, [rows, dcol])
                prod = kvv * qv
                acc = prod if acc is None else acc + prod
            sc = acc * INV_SQRT_D
            sc = jnp.minimum(jnp.maximum(sc, -5.0), 5.0)
            p = jnp.exp(sc)
            plsc.store_scatter(
                w_b, [rows, jnp.full((16,), HD + hh, jnp.int32)], p)
            for r in range(D):
                dcol = col0 + jnp.bitwise_xor(lanes, r)
                vv = plsc.load_gather(kv_b, [rows, HD + dcol])
                plsc.store_scatter(w_b, [rows, dcol], vv * p)
            return carry2

        lax.fori_loop(0, H, _head, 0, unroll=4)
        return carry

    lax.fori_loop(0, C // 16, _group, 0)


def _edge_body(acc_rows, ch,
               q_hbm, kv_hbm, pk_hbm, zeros_hbm, out_hbm,
               acc_s, pkblk, kvbuf, qbuf, wbuf, gidx, sidx, sems):
    c = lax.axis_index("c")
    s = lax.axis_index("s")
    w = s * NC + c

    sem_kv = [sems.at[0, b] for b in range(NBUF)]
    sem_q = [sems.at[1, b] for b in range(NBUF)]
    sem_w = [sems.at[2, b] for b in range(NBUF)]

    rpt = acc_rows // NS
    r0 = s * rpt
    pltpu.sync_copy(zeros_hbm.at[pl.ds(r0, rpt)], acc_s.at[pl.ds(r0, rpt)])

    row0 = w * ch
    pltpu.sync_copy(pk_hbm.at[pl.ds(row0, ch + NBUF)], pkblk)

    plsc.subcore_barrier()

    def _unpack_gather_idx(slot, local_row):
        for g in range(C // 16):
            pkv = pkblk[local_row, pl.ds(g * 16, 16)]
            gidx[0, slot, pl.ds(g * 16, 16)] = jnp.bitwise_and(
                pkv, jnp.int32(0xFFFF))
            gidx[1, slot, pl.ds(g * 16, 16)] = lax.shift_right_logical(
                pkv, jnp.int32(16))

    def _issue(slot, local_row):
        _unpack_gather_idx(slot, local_row)
        pltpu.async_copy(
            kv_hbm.at[gidx.at[0, slot]], kvbuf.at[slot], sem_kv[slot])
        pltpu.async_copy(
            q_hbm.at[gidx.at[1, slot]], qbuf.at[slot], sem_q[slot])

    def _drain(slot):
        pltpu.make_async_copy(
            kv_hbm.at[gidx.at[0, slot]], kvbuf.at[slot],
            sem_kv[slot]).wait()
        pltpu.make_async_copy(
            q_hbm.at[gidx.at[1, slot]], qbuf.at[slot], sem_q[slot]).wait()

    def _wait_scatter(slot):
        pltpu.make_async_copy(
            wbuf.at[slot], acc_s.at[sidx.at[slot]], sem_w[slot]).wait()

    for b in range(NBUF):
        _issue(b, b)

    def _round(t, carry):
        jj = NBUF * t
        for b in range(NBUF):
            _drain(b)

            @pl.when(t > 0)
            def _():
                _wait_scatter(b)

            _compute_chunk(kvbuf.at[b], qbuf.at[b], wbuf.at[b])
            for g in range(C // 16):
                pkv = pkblk[jj + b, pl.ds(g * 16, 16)]
                sidx[b, pl.ds(g * 16, 16)] = lax.shift_right_logical(
                    pkv, jnp.int32(16))
            pltpu.async_copy(
                wbuf.at[b], acc_s.at[sidx.at[b]], sem_w[b], add=True)
            _issue(b, jj + b + NBUF)
        return carry

    lax.fori_loop(0, ch // NBUF, _round, 0)

    for b in range(NBUF):
        _drain(b)
        _wait_scatter(b)

    plsc.subcore_barrier()
    pltpu.sync_copy(acc_s.at[pl.ds(r0, rpt)], out_hbm.at[c, pl.ds(r0, rpt)])


def _edge_phase(q, kv, pk2, zeros, acc_rows, ch):
    mesh = plsc.VectorSubcoreMesh(core_axis_name="c", subcore_axis_name="s")
    body = functools.partial(_edge_body, acc_rows, ch)
    return pl.kernel(
        body,
        out_type=jax.ShapeDtypeStruct((NC, acc_rows, ROW), jnp.float32),
        mesh=mesh,
        scratch_types=[
            pltpu.VMEM_SHARED((acc_rows, ROW), jnp.float32),
            pltpu.VMEM((ch + NBUF, C), jnp.int32),
            pltpu.VMEM((NBUF, C, 2 * HD), jnp.float32),
            pltpu.VMEM((NBUF, C, HD), jnp.float32),
            pltpu.VMEM((NBUF, C, ROW), jnp.float32),
            pltpu.VMEM((2, NBUF, C), jnp.int32),
            pltpu.VMEM((NBUF, C), jnp.int32),
            pltpu.SemaphoreType.DMA((3, NBUF)),
        ],
        compiler_params=pltpu.CompilerParams(
            use_tc_tiling_on_sc=False, needs_layout_passes=False),
    )(q, kv, pk2, zeros)


def _combine_body(acc_ref, out_ref):
    ab = acc_ref[...]
    a = ab[0] + ab[1]
    wv = a[:, :HD]
    z = a[:, HD:HD + H]
    rowi = lax.broadcasted_iota(jnp.int32, (H, HD), 0)
    coli = lax.broadcasted_iota(jnp.int32, (H, HD), 1)
    bmat = (coli // D == rowi).astype(jnp.float32)
    zrep = jnp.dot(z, bmat, preferred_element_type=jnp.float32)
    out_ref[...] = wv / zrep


def _combine(acc, n_out, blk):
    grid = n_out // blk
    return pl.pallas_call(
        _combine_body,
        grid=(grid,),
        in_specs=[pl.BlockSpec((NC, blk, ROW), lambda i: (0, i, 0))],
        out_specs=pl.BlockSpec((blk, HD), lambda i: (i, 0)),
        out_shape=jax.ShapeDtypeStruct((n_out, HD), jnp.float32),
    )(acc)


def kernel(h, edge_index, WQ, WK, WV):
    n, in_dim = h.shape
    e = edge_index.shape[1]

    wt = jnp.concatenate([WQ.T, WK.T, WV.T], axis=1)
    blk_n = 1000 if n % 1000 == 0 else 8
    n_pad = _cdiv(n, blk_n) * blk_n
    h_p = h if n_pad == n else jnp.pad(h, ((0, n_pad - n), (0, 0)))

    src = edge_index[0]
    dst = edge_index[1]
    ch = NBUF * _cdiv(e, NW * C * NBUF)
    rows_total = NW * ch + NBUF
    e_pad = rows_total * C
    src = jnp.concatenate([src, jnp.zeros((e_pad - e,), jnp.int32)])
    dst = jnp.concatenate([dst, jnp.full((e_pad - e,), n, jnp.int32)])
    pk = jnp.bitwise_or(src, jnp.left_shift(dst, 16))
    pk2 = pk.reshape(rows_total, C)

    acc_rows = max(n + 1, _cdiv(n, 1000) * 1000)
    acc_rows = NS * _cdiv(acc_rows, NS)
    zeros = jnp.zeros((acc_rows, ROW), jnp.float32)

    q, kv = _qkv(h_p, wt, blk_n)
    q = q[:n] if n_pad != n else q
    kv = kv[:n] if n_pad != n else kv
    acc = _edge_phase(q, kv, pk2, zeros, acc_rows, ch)
    blk_o = 1000 if n % 1000 == 0 else 8
    n_out = _cdiv(n, blk_o) * blk_o
    out = _combine(acc, n_out, blk_o)
    return out[:n].reshape(n, H, D)
